# Initial kernel scaffold; baseline (speedup 1.0000x reference)
import jax
import jax.numpy as jnp
from jax.experimental import pallas as pl


def kernel(X, gene_mask, token_ids, technology_mean):
    raise NotImplementedError("write your pallas kernel here")


# speedup vs baseline: 14.7921x; 14.7921x over previous
import functools

import jax
import jax.numpy as jnp
from jax.experimental import pallas as pl

_SEQ_LEN = 1500
_AUX = 30


def _sort_body(n2, seq_len, xt_ref, p_ref, o_ref):
    G, R = xt_ref.shape
    key = xt_ref[...]
    pays = p_ref[...]

    pad = n2 - G
    key = jnp.concatenate(
        [key, jnp.full((pad, R), -1.0, jnp.float32)], axis=0)
    pays = jnp.concatenate(
        [pays, jnp.full((pad, R), (n2 - 1) << 16, jnp.int32)], axis=0)

    log_n = n2.bit_length() - 1
    for k in range(1, log_n + 1):
        for j in range(k - 1, -1, -1):
            d = 1 << j
            mseg = n2 // (2 * d)
            w = d * R
            kr = key.reshape(mseg, 2 * w)
            pr = pays.reshape(mseg, 2 * w)
            ka, kb = kr[:, :w], kr[:, w:]
            pa, pb = pr[:, :w], pr[:, w:]
            swap_desc = (ka < kb) | ((ka == kb) & (pa > pb))
            if k == log_n:
                swap = swap_desc
            else:
                sidx = jax.lax.broadcasted_iota(jnp.int32, (mseg, 1), 0)
                desc_m = ((sidx >> (k - 1 - j)) & 1) == 0
                swap_asc = (ka > kb) | ((ka == kb) & (pa < pb))
                swap = (swap_desc & desc_m) | (swap_asc & ~desc_m)
            na = jnp.where(swap, kb, ka)
            nb = jnp.where(swap, ka, kb)
            npa = jnp.where(swap, pb, pa)
            npb = jnp.where(swap, pa, pb)
            key = jnp.concatenate([na, nb], axis=1).reshape(n2, R)
            pays = jnp.concatenate([npa, npb], axis=1).reshape(n2, R)

    o_ref[...] = pays[:seq_len, :] & 0xFFFF


def kernel(X, gene_mask, token_ids, technology_mean):
    N, G = X.shape
    n2 = 1 << (max(G, _SEQ_LEN) - 1).bit_length()
    R = 128
    grid = N // R

    tech = jnp.nan_to_num(technology_mean)
    tech = tech + (tech == 0).astype(jnp.float32)
    tech_g = jnp.take(tech, token_ids)
    w = jnp.where(gene_mask, 1.0 / tech_g, 0.0)
    xt = (X * w[None, :]).T

    idx = jnp.arange(G, dtype=jnp.int32)
    payload = (idx << 16) | (token_ids.astype(jnp.int32) + _AUX)
    pay2d = jnp.broadcast_to(payload[:, None], (G, R)).astype(jnp.int32)

    out_t = pl.pallas_call(
        functools.partial(_sort_body, n2, _SEQ_LEN),
        grid=(grid,),
        in_specs=[
            pl.BlockSpec((G, R), lambda i: (0, i)),
            pl.BlockSpec((G, R), lambda i: (0, 0)),
        ],
        out_specs=pl.BlockSpec((_SEQ_LEN, R), lambda i: (0, i)),
        out_shape=jax.ShapeDtypeStruct((_SEQ_LEN, N), jnp.int32),
    )(xt, pay2d)
    return out_t.T


# speedup vs baseline: 26.0867x; 1.7636x over previous
import functools

import jax
import jax.numpy as jnp
from jax import lax
from jax.experimental import pallas as pl
from jax.experimental.pallas import tpu as pltpu
from jax.experimental.pallas import tpu_sc as plsc

_SEQ = 1500
_AUX = 30
_N2 = 2048
_NVREG = _N2 // 16
_OUTP = 1504


def _vsort(kref, pref, v, desc):
    s = v * 16
    ks, ps = plsc.sort_key_val(kref[pl.ds(s, 16)], pref[pl.ds(s, 16)],
                               descending=desc)
    kref[pl.ds(s, 16)] = ks
    pref[pl.ds(s, 16)] = ps


def _pair(kref, pref, va, vb, desc):
    sa, sb = va * 16, vb * 16
    ka = kref[pl.ds(sa, 16)]
    kb = kref[pl.ds(sb, 16)]
    pa = pref[pl.ds(sa, 16)]
    pb = pref[pl.ds(sb, 16)]
    swap = (ka < kb) if desc else (ka > kb)
    kref[pl.ds(sa, 16)] = jnp.where(swap, kb, ka)
    kref[pl.ds(sb, 16)] = jnp.where(swap, ka, kb)
    pref[pl.ds(sa, 16)] = jnp.where(swap, pb, pa)
    pref[pl.ds(sb, 16)] = jnp.where(swap, pa, pb)


def _sc_body(rows_per_w, gp, xw_hbm, tmpl_hbm, out_hbm,
             key_v, pay_v, tmpl_v, out_v):
    nc = 2
    wid = lax.axis_index("s") * nc + lax.axis_index("c")
    pltpu.sync_copy(tmpl_hbm, tmpl_v)

    def row_body(i, _):
        row = wid * rows_per_w + i
        pltpu.sync_copy(xw_hbm.at[row], key_v.at[pl.ds(0, gp)])

        neg1 = jnp.full((16,), -1.0, jnp.float32)

        def pad_body(v, _):
            key_v[pl.ds(gp + v * 16, 16)] = neg1
            return _
        lax.fori_loop(0, (_N2 - gp) // 16, pad_body, None)

        def tmpl_body(v, _):
            pay_v[pl.ds(v * 16, 16)] = tmpl_v[pl.ds(v * 16, 16)]
            return _
        lax.fori_loop(0, _NVREG, tmpl_body, None)

        def init_body(v2, _):
            _vsort(key_v, pay_v, v2 * 2, True)
            _vsort(key_v, pay_v, v2 * 2 + 1, False)
            return _
        lax.fori_loop(0, _NVREG // 2, init_body, None)

        for k in range(5, 12):
            for j in range(k - 1, 3, -1):
                dd = 1 << (j - 4)

                def mk_va(p, j=j, dd=dd):
                    return ((p >> (j - 4)) << (j - 3)) + (p & (dd - 1))

                if k == 11:
                    def pair_all(p, _, dd=dd, mk_va=mk_va):
                        va = mk_va(p)
                        _pair(key_v, pay_v, va, va + dd, True)
                        return _
                    lax.fori_loop(0, _NVREG // 2, pair_all, None)
                else:
                    bb = 1 << (k - 5)

                    def outer(b, _, dd=dd, bb=bb, mk_va=mk_va):
                        def inner(w, _):
                            p = b * 2 * bb + w
                            va = mk_va(p)
                            _pair(key_v, pay_v, va, va + dd, True)
                            va2 = mk_va(p + bb)
                            _pair(key_v, pay_v, va2, va2 + dd, False)
                            return _
                        return lax.fori_loop(0, bb, inner, _)
                    lax.fori_loop(0, 1 << (10 - k), outer, None)

            if k == 11:
                def cl_all(v, _):
                    _vsort(key_v, pay_v, v, True)
                    return _
                lax.fori_loop(0, _NVREG, cl_all, None)
            else:
                bv = 1 << (k - 4)

                def cl_outer(b, _, bv=bv):
                    def cl_inner(w, _):
                        _vsort(key_v, pay_v, b * 2 * bv + w, True)
                        _vsort(key_v, pay_v, b * 2 * bv + bv + w, False)
                        return _
                    return lax.fori_loop(0, bv, cl_inner, _)
                lax.fori_loop(0, 1 << (10 - k), cl_outer, None)

        def ext_body(v, _):
            out_v[pl.ds(v * 16, 16)] = pay_v[pl.ds(v * 16, 16)] & 0xFFFF
            return _
        lax.fori_loop(0, _OUTP // 16, ext_body, None)

        pltpu.sync_copy(out_v, out_hbm.at[row])
        return _

    lax.fori_loop(0, rows_per_w, row_body, None)


def kernel(X, gene_mask, token_ids, technology_mean):
    N, G = X.shape
    gp = ((G + 15) // 16) * 16

    tech = jnp.nan_to_num(technology_mean)
    tech = tech + (tech == 0).astype(jnp.float32)
    tech_g = jnp.take(tech, token_ids)
    w = jnp.where(gene_mask, 1.0 / tech_g, 0.0)
    xw = X * w[None, :]
    xw_p = jnp.concatenate(
        [xw, jnp.full((N, gp - G), -1.0, jnp.float32)], axis=1)

    idx = jnp.arange(G, dtype=jnp.int32)
    tmpl_g = (idx << 16) | (token_ids.astype(jnp.int32) + _AUX)
    tmpl = jnp.concatenate(
        [tmpl_g, jnp.full((_N2 - G,), (_N2 - 1) << 16, jnp.int32)])

    mesh = plsc.VectorSubcoreMesh(core_axis_name="c", subcore_axis_name="s")
    nw = 32
    rows_per_w = N // nw

    sc = functools.partial(
        pl.kernel,
        out_type=jax.ShapeDtypeStruct((N, _OUTP), jnp.int32),
        mesh=mesh,
        compiler_params=pltpu.CompilerParams(
            needs_layout_passes=False, use_tc_tiling_on_sc=False),
        scratch_types=[
            pltpu.VMEM((_N2,), jnp.float32),
            pltpu.VMEM((_N2,), jnp.int32),
            pltpu.VMEM((_N2,), jnp.int32),
            pltpu.VMEM((_OUTP,), jnp.int32),
        ],
    )(functools.partial(_sc_body, rows_per_w, gp))

    out_p = sc(xw_p, tmpl)
    return out_p[:, :_SEQ]


# speedup vs baseline: 49.8536x; 1.9111x over previous
import functools

import jax
import jax.numpy as jnp
from jax import lax
from jax.experimental import pallas as pl
from jax.experimental.pallas import tpu as pltpu
from jax.experimental.pallas import tpu_sc as plsc

_SEQ = 1500
_AUX = 30
_N2 = 2048
_NVREG = _N2 // 16
_OUTP = 1504


def _cmpx(K, P, i, l, desc):
    ka, kb, pa, pb = K[i], K[l], P[i], P[l]
    swap = (ka < kb) if desc else (ka > kb)
    K[i] = jnp.where(swap, kb, ka)
    K[l] = jnp.where(swap, ka, kb)
    P[i] = jnp.where(swap, pb, pa)
    P[l] = jnp.where(swap, pa, pb)


def _load_group(kref, pref, base, gs):
    K = [kref[pl.ds((base + i) * 16, 16)] for i in range(gs)]
    P = [pref[pl.ds((base + i) * 16, 16)] for i in range(gs)]
    return K, P


def _store_group(kref, pref, base, K, P):
    for i in range(len(K)):
        kref[pl.ds((base + i) * 16, 16)] = K[i]
        pref[pl.ds((base + i) * 16, 16)] = P[i]


def _init_group(kref, pref, g, desc):
    base = g * 4
    K, P = _load_group(kref, pref, base, 4)
    for i in range(4):
        K[i], P[i] = plsc.sort_key_val(K[i], P[i], descending=(i % 2 == 0))
    _cmpx(K, P, 0, 1, True)
    _cmpx(K, P, 2, 3, False)
    for i in range(4):
        K[i], P[i] = plsc.sort_key_val(K[i], P[i], descending=(i < 2))
    _cmpx(K, P, 0, 2, desc)
    _cmpx(K, P, 1, 3, desc)
    _cmpx(K, P, 0, 1, desc)
    _cmpx(K, P, 2, 3, desc)
    for i in range(4):
        K[i], P[i] = plsc.sort_key_val(K[i], P[i], descending=desc)
    _store_group(kref, pref, base, K, P)


def _tail_group(kref, pref, g, desc):
    base = g * 8
    K, P = _load_group(kref, pref, base, 8)
    for i in range(4):
        _cmpx(K, P, i, i + 4, desc)
    for i in (0, 1, 4, 5):
        _cmpx(K, P, i, i + 2, desc)
    for i in (0, 2, 4, 6):
        _cmpx(K, P, i, i + 1, desc)
    for i in range(8):
        K[i], P[i] = plsc.sort_key_val(K[i], P[i], descending=desc)
    _store_group(kref, pref, base, K, P)


def _pair(kref, pref, va, vb, desc):
    sa, sb = va * 16, vb * 16
    K = [kref[pl.ds(sa, 16)], kref[pl.ds(sb, 16)]]
    P = [pref[pl.ds(sa, 16)], pref[pl.ds(sb, 16)]]
    _cmpx(K, P, 0, 1, desc)
    kref[pl.ds(sa, 16)] = K[0]
    kref[pl.ds(sb, 16)] = K[1]
    pref[pl.ds(sa, 16)] = P[0]
    pref[pl.ds(sb, 16)] = P[1]


def _split_loop(n, bb, body):
    if bb >= n:
        def all_body(i, _):
            body(i, True)
            return _
        lax.fori_loop(0, n, all_body, None)
    else:
        def outer(b, _):
            def inner(w, _):
                body(b * 2 * bb + w, True)
                body(b * 2 * bb + bb + w, False)
                return _
            return lax.fori_loop(0, bb, inner, _)
        lax.fori_loop(0, n // (2 * bb), outer, None)


def _sc_body(rows_per_w, gp, xw_hbm, tmpl_hbm, out_hbm,
             key_v, pay_v, tmpl_v, out_v):
    nc = 2
    wid = lax.axis_index("s") * nc + lax.axis_index("c")
    pltpu.sync_copy(tmpl_hbm, tmpl_v)

    def row_body(i, _):
        row = wid * rows_per_w + i
        pltpu.sync_copy(xw_hbm.at[row], key_v.at[pl.ds(0, gp)])

        neg1 = jnp.full((16,), -1.0, jnp.float32)

        def pad_body(v, _):
            key_v[pl.ds(gp + v * 16, 16)] = neg1
            return _
        lax.fori_loop(0, (_N2 - gp) // 16, pad_body, None)

        def tmpl_body(v, _):
            pay_v[pl.ds(v * 16, 16)] = tmpl_v[pl.ds(v * 16, 16)]
            return _
        lax.fori_loop(0, _NVREG, tmpl_body, None)

        _split_loop(_NVREG // 4, 1,
                    lambda g, d: _init_group(key_v, pay_v, g, d))

        for k in range(7, 12):
            for j in range(k - 1, 6, -1):
                dd = 1 << (j - 4)

                def pair_level(p, d, j=j, dd=dd):
                    va = ((p >> (j - 4)) << (j - 3)) + (p & (dd - 1))
                    _pair(key_v, pay_v, va, va + dd, d)

                _split_loop(_NVREG // 2, 1 << (k - 5), pair_level)

            _split_loop(_NVREG // 8, 1 << (k - 7),
                        lambda g, d: _tail_group(key_v, pay_v, g, d))

        def ext_body(v, _):
            out_v[pl.ds(v * 16, 16)] = pay_v[pl.ds(v * 16, 16)] & 0xFFFF
            return _
        lax.fori_loop(0, _OUTP // 16, ext_body, None)

        pltpu.sync_copy(out_v, out_hbm.at[row])
        return _

    lax.fori_loop(0, rows_per_w, row_body, None)


def kernel(X, gene_mask, token_ids, technology_mean):
    N, G = X.shape
    gp = ((G + 15) // 16) * 16

    tech = jnp.nan_to_num(technology_mean)
    tech = tech + (tech == 0).astype(jnp.float32)
    tech_g = jnp.take(tech, token_ids)
    w = jnp.where(gene_mask, 1.0 / tech_g, 0.0)
    xw = X * w[None, :]
    xw_p = jnp.concatenate(
        [xw, jnp.full((N, gp - G), -1.0, jnp.float32)], axis=1)

    idx = jnp.arange(G, dtype=jnp.int32)
    tmpl_g = (idx << 16) | (token_ids.astype(jnp.int32) + _AUX)
    tmpl = jnp.concatenate(
        [tmpl_g, jnp.full((_N2 - G,), (_N2 - 1) << 16, jnp.int32)])

    mesh = plsc.VectorSubcoreMesh(core_axis_name="c", subcore_axis_name="s")
    nw = 32
    rows_per_w = N // nw

    sc = functools.partial(
        pl.kernel,
        out_type=jax.ShapeDtypeStruct((N, _OUTP), jnp.int32),
        mesh=mesh,
        compiler_params=pltpu.CompilerParams(
            needs_layout_passes=False, use_tc_tiling_on_sc=False),
        scratch_types=[
            pltpu.VMEM((_N2,), jnp.float32),
            pltpu.VMEM((_N2,), jnp.int32),
            pltpu.VMEM((_N2,), jnp.int32),
            pltpu.VMEM((_OUTP,), jnp.int32),
        ],
    )(functools.partial(_sc_body, rows_per_w, gp))

    out_p = sc(xw_p, tmpl)
    return out_p[:, :_SEQ]


# speedup vs baseline: 65.7555x; 1.3190x over previous
import functools

import jax
import jax.numpy as jnp
from jax import lax
from jax.experimental import pallas as pl
from jax.experimental.pallas import tpu as pltpu
from jax.experimental.pallas import tpu_sc as plsc

_SEQ = 1500
_AUX = 30
_N2 = 2048
_NVREG = _N2 // 16
_OUTP = 1504


def _cmpx(K, P, i, l, desc):
    ka, kb, pa, pb = K[i], K[l], P[i], P[l]
    swap = (ka < kb) if desc else (ka > kb)
    K[i] = jnp.where(swap, kb, ka)
    K[l] = jnp.where(swap, ka, kb)
    P[i] = jnp.where(swap, pb, pa)
    P[l] = jnp.where(swap, pa, pb)


def _load_group(kref, pref, base, gs):
    K = [kref[pl.ds((base + i) * 16, 16)] for i in range(gs)]
    P = [pref[pl.ds((base + i) * 16, 16)] for i in range(gs)]
    return K, P


def _store_group(kref, pref, base, K, P):
    for i in range(len(K)):
        kref[pl.ds((base + i) * 16, 16)] = K[i]
        pref[pl.ds((base + i) * 16, 16)] = P[i]


def _init_group(kref, pref, tref, g, desc):
    base = g * 4
    K, _ = _load_group(kref, kref, base, 4)
    _, P = _load_group(tref, tref, base, 4)
    for i in range(4):
        K[i], P[i] = plsc.sort_key_val(K[i], P[i], descending=(i % 2 == 0))
    _cmpx(K, P, 0, 1, True)
    _cmpx(K, P, 2, 3, False)
    for i in range(4):
        K[i], P[i] = plsc.sort_key_val(K[i], P[i], descending=(i < 2))
    _cmpx(K, P, 0, 2, desc)
    _cmpx(K, P, 1, 3, desc)
    _cmpx(K, P, 0, 1, desc)
    _cmpx(K, P, 2, 3, desc)
    for i in range(4):
        K[i], P[i] = plsc.sort_key_val(K[i], P[i], descending=desc)
    _store_group(kref, pref, base, K, P)


def _tail_group(kref, pref, g, desc):
    base = g * 8
    K, P = _load_group(kref, pref, base, 8)
    for i in range(4):
        _cmpx(K, P, i, i + 4, desc)
    for i in (0, 1, 4, 5):
        _cmpx(K, P, i, i + 2, desc)
    for i in (0, 2, 4, 6):
        _cmpx(K, P, i, i + 1, desc)
    for i in range(8):
        K[i], P[i] = plsc.sort_key_val(K[i], P[i], descending=desc)
    _store_group(kref, pref, base, K, P)


def _tail16_group(kref, pref, g, desc):
    base = g * 16
    K, P = _load_group(kref, pref, base, 16)
    for i in range(8):
        _cmpx(K, P, i, i + 8, desc)
    for h in (0, 8):
        for i in range(4):
            _cmpx(K, P, h + i, h + i + 4, desc)
    for h in (0, 4, 8, 12):
        for i in range(2):
            _cmpx(K, P, h + i, h + i + 2, desc)
    for i in range(0, 16, 2):
        _cmpx(K, P, i, i + 1, desc)
    for i in range(16):
        K[i], P[i] = plsc.sort_key_val(K[i], P[i], descending=desc)
    _store_group(kref, pref, base, K, P)


def _pair(kref, pref, va, vb, desc):
    sa, sb = va * 16, vb * 16
    K = [kref[pl.ds(sa, 16)], kref[pl.ds(sb, 16)]]
    P = [pref[pl.ds(sa, 16)], pref[pl.ds(sb, 16)]]
    _cmpx(K, P, 0, 1, desc)
    kref[pl.ds(sa, 16)] = K[0]
    kref[pl.ds(sb, 16)] = K[1]
    pref[pl.ds(sa, 16)] = P[0]
    pref[pl.ds(sb, 16)] = P[1]


def _split_loop(n, bb, body):
    if bb >= n:
        def all_body(i, _):
            body(i, True)
            return _
        lax.fori_loop(0, n, all_body, None)
    else:
        def outer(b, _):
            def inner(w, _):
                body(b * 2 * bb + w, True)
                body(b * 2 * bb + bb + w, False)
                return _
            return lax.fori_loop(0, bb, inner, _)
        lax.fori_loop(0, n // (2 * bb), outer, None)


def _sc_body(rows_per_w, gp, xw_hbm, tmpl_hbm, out_hbm,
             key_v, pay_v, tmpl_v, out_v):
    nc = 2
    wid = lax.axis_index("s") * nc + lax.axis_index("c")
    pltpu.sync_copy(tmpl_hbm, tmpl_v)

    gpv = gp // 16
    n_init = -(-gpv // 4)
    neg1 = jnp.full((16,), -1.0, jnp.float32)
    padp = jnp.full((16,), (_N2 - 1) << 16, jnp.int32)

    def row_body(i, _):
        row = wid * rows_per_w + i
        pltpu.sync_copy(xw_hbm.at[row], key_v.at[pl.ds(0, gp)])

        for v in range(gpv, n_init * 4):
            key_v[pl.ds(v * 16, 16)] = neg1

        _split_loop(n_init, 1,
                    lambda g, d: _init_group(key_v, pay_v, tmpl_v, g, d))

        def padg_body(v, _):
            key_v[pl.ds(v * 16, 16)] = neg1
            pay_v[pl.ds(v * 16, 16)] = padp
            return _
        lax.fori_loop(n_init * 4, _NVREG, padg_body, None)

        for k in range(7, 12):
            if k == 7:
                _split_loop(_NVREG // 8, 1,
                            lambda g, d: _tail_group(key_v, pay_v, g, d))
                continue
            for j in range(k - 1, 7, -1):
                dd = 1 << (j - 4)

                def pair_level(p, d, j=j, dd=dd):
                    va = ((p >> (j - 4)) << (j - 3)) + (p & (dd - 1))
                    _pair(key_v, pay_v, va, va + dd, d)

                _split_loop(_NVREG // 2, 1 << (k - 5), pair_level)

            _split_loop(_NVREG // 16, 1 << (k - 8),
                        lambda g, d: _tail16_group(key_v, pay_v, g, d))

        def ext_body(v, _):
            out_v[pl.ds(v * 16, 16)] = pay_v[pl.ds(v * 16, 16)] & 0xFFFF
            return _
        lax.fori_loop(0, _OUTP // 16, ext_body, None)

        pltpu.sync_copy(out_v, out_hbm.at[row])
        return _

    lax.fori_loop(0, rows_per_w, row_body, None)


def kernel(X, gene_mask, token_ids, technology_mean):
    N, G = X.shape
    gp = ((G + 15) // 16) * 16

    tech = jnp.nan_to_num(technology_mean)
    tech = tech + (tech == 0).astype(jnp.float32)
    tech_g = jnp.take(tech, token_ids)
    w = jnp.where(gene_mask, 1.0 / tech_g, 0.0)
    xw = X * w[None, :]
    xw_p = jnp.concatenate(
        [xw, jnp.full((N, gp - G), -1.0, jnp.float32)], axis=1)

    idx = jnp.arange(G, dtype=jnp.int32)
    tmpl_g = (idx << 16) | (token_ids.astype(jnp.int32) + _AUX)
    tmpl = jnp.concatenate(
        [tmpl_g, jnp.full((_N2 - G,), (_N2 - 1) << 16, jnp.int32)])

    mesh = plsc.VectorSubcoreMesh(core_axis_name="c", subcore_axis_name="s")
    nw = 32
    rows_per_w = N // nw

    sc = functools.partial(
        pl.kernel,
        out_type=jax.ShapeDtypeStruct((N, _OUTP), jnp.int32),
        mesh=mesh,
        compiler_params=pltpu.CompilerParams(
            needs_layout_passes=False, use_tc_tiling_on_sc=False),
        scratch_types=[
            pltpu.VMEM((_N2,), jnp.float32),
            pltpu.VMEM((_N2,), jnp.int32),
            pltpu.VMEM((_N2,), jnp.int32),
            pltpu.VMEM((_OUTP,), jnp.int32),
        ],
    )(functools.partial(_sc_body, rows_per_w, gp))

    out_p = sc(xw_p, tmpl)
    return out_p[:, :_SEQ]
